# Initial kernel scaffold; baseline (speedup 1.0000x reference)
import jax
import jax.numpy as jnp
from jax.experimental import pallas as pl


def kernel(img, cloud, img_tar, cloud_tar, current_feat, target_feat, w_conv1, b_conv1, w_conv2, b_conv2, w_pconv1, b_pconv1, w_pconv2, b_pconv2, w_fc1, b_fc1, w_fc2, b_fc2, w_fuse2, b_fuse2, w_pn1, b_pn1, w_pn2, b_pn2, w_pn3, b_pn3):
    raise NotImplementedError("write your pallas kernel here")


# speedup vs baseline: 2.3873x; 2.3873x over previous
import functools

import jax
import jax.numpy as jnp
from jax import lax
from jax.experimental import pallas as pl
from jax.experimental.pallas import tpu as pltpu
from jax.experimental.pallas import tpu_sc as plsc

N = 500
NPTS = 512
K1 = 12
KP = 4
NEG = -3.0e38

_NC = 2
_NS = 16
_NW = _NC * _NS
_QPW = NPTS // _NW
_IPW = _QPW * K1
_HALF = _IPW // 2

_DOT = dict(precision=lax.Precision.HIGHEST, preferred_element_type=jnp.float32)


def _lrelu(x):
    return jnp.where(x >= 0, x, 0.01 * x)


def _mlp(x, w1t, b1, w2t, b2):
    h = lax.dot_general(x, w1t, (((1,), (0,)), ((), ())), **_DOT) + b1
    return lax.dot_general(_lrelu(h), w2t, (((1,), (0,)), ((), ())), **_DOT) + b2


def _dist2(u, v):
    return lax.dot_general(u, v, (((1,), (1,)), ((), ())), **_DOT)


def _topk(d, k):
    iota = lax.broadcasted_iota(jnp.int32, (NPTS, NPTS), 1)
    idxs, vals = [], []
    work = d
    for _ in range(k):
        m = jnp.min(work, axis=1, keepdims=True)
        hit = work == m
        a = jnp.min(jnp.where(hit, iota, NPTS), axis=1, keepdims=True)
        idxs.append(a)
        vals.append(m)
        work = jnp.where(iota == a, 3.0e38, work)
    return jnp.concatenate(idxs, 1), jnp.concatenate(vals, 1)


def _softw(vals):
    nrm = jnp.sqrt(jnp.maximum(vals, 1e-12))
    rmask = lax.broadcasted_iota(jnp.int32, (NPTS, K1), 0) < N
    logit = jnp.where(rmask, -nrm, NEG)
    e = jnp.exp(logit - jnp.max(logit))
    return e / jnp.sum(e)


def _tc_a_body(pa_ref, pb_ref, xa_ref, xb_ref,
               w1c_ref, b1c_ref, w2c_ref, b2c_ref,
               w1p_ref, b1p_ref, w2p_ref, b2p_ref,
               ifa_ref, ifb_ref, pfa_ref, pfb_ref,
               idx1_ref, w1_ref, idx2_ref, w2_ref, g_ref):
    pa = pa_ref[...]
    pb = pb_ref[...]
    sa = jnp.sum(pa * pa, axis=1, keepdims=True)
    sb = jnp.sum(pb * pb, axis=1, keepdims=True)
    one = jnp.ones((NPTS, 1), jnp.float32)
    ua = jnp.concatenate([-2.0 * pa, sa, one], 1)
    ub = jnp.concatenate([-2.0 * pb, sb, one], 1)
    va = jnp.concatenate([pa, one, sa], 1)
    vb = jnp.concatenate([pb, one, sb], 1)
    d1 = _dist2(ua, vb)
    d2 = _dist2(ua, va)
    d1t = _dist2(ub, va)

    idx1, v1 = _topk(d1, K1)
    idx2, v2 = _topk(d2, K1)
    idxp, _ = _topk(d1t, KP)

    idx1_ref[...] = idx1
    idx2_ref[...] = idx2
    w1_ref[...] = _softw(v1)
    w2_ref[...] = _softw(v2)

    giota = lax.broadcasted_iota(jnp.int32, (NPTS, NPTS), 1)
    g = jnp.zeros((NPTS, NPTS), jnp.float32)
    for i in range(KP):
        g = g + (idxp[:, i:i + 1] == giota).astype(jnp.float32)
    g_ref[...] = g

    w1c, b1c = w1c_ref[...], b1c_ref[...]
    w2c, b2c = w2c_ref[...], b2c_ref[...]
    w1p, b1p = w1p_ref[...], b1p_ref[...]
    w2p, b2p = w2p_ref[...], b2p_ref[...]
    ifa_ref[...] = _mlp(xa_ref[...], w1c, b1c, w2c, b2c)
    ifb_ref[...] = _mlp(xb_ref[...], w1c, b1c, w2c, b2c)
    pfa_ref[...] = _mlp(pa[:, :3], w1p, b1p, w2p, b2p)
    pfb_ref[...] = _mlp(pb[:, :3], w1p, b1p, w2p, b2p)


def _sc_gather_body(taba_ref, tabb_ref, idx_ref,
                    outa_ref, outb_ref,
                    idxv0_ref, idxv1_ref, bufa_ref, bufb_ref, sem):
    wid = lax.axis_index("s") * _NC + lax.axis_index("c")
    base = wid * _IPW
    pltpu.sync_copy(idx_ref.at[pl.ds(base, _HALF)], idxv0_ref)
    pltpu.sync_copy(idx_ref.at[pl.ds(base + _HALF, _HALF)], idxv1_ref)
    pltpu.async_copy(taba_ref.at[idxv0_ref], bufa_ref, sem).wait()
    pltpu.sync_copy(bufa_ref, outa_ref.at[pl.ds(base, _HALF)])
    pltpu.async_copy(taba_ref.at[idxv1_ref], bufa_ref, sem).wait()
    pltpu.sync_copy(bufa_ref, outa_ref.at[pl.ds(base + _HALF, _HALF)])
    pltpu.async_copy(tabb_ref.at[idxv0_ref], bufb_ref, sem).wait()
    pltpu.sync_copy(bufb_ref, outb_ref.at[pl.ds(base, _HALF)])
    pltpu.async_copy(tabb_ref.at[idxv1_ref], bufb_ref, sem).wait()
    pltpu.sync_copy(bufb_ref, outb_ref.at[pl.ds(base + _HALF, _HALF)])


@functools.cache
def _sc_gather():
    return functools.partial(
        pl.kernel,
        out_type=[jax.ShapeDtypeStruct((NPTS * K1, 128), jnp.float32)] * 2,
        mesh=plsc.VectorSubcoreMesh(core_axis_name="c", subcore_axis_name="s",
                                    num_cores=_NC, num_subcores=_NS),
        scratch_types=[
            pltpu.VMEM((_HALF,), jnp.int32),
            pltpu.VMEM((_HALF,), jnp.int32),
            pltpu.VMEM((_HALF, 128), jnp.float32),
            pltpu.VMEM((_HALF, 128), jnp.float32),
            pltpu.SemaphoreType.DMA,
        ],
    )(_sc_gather_body)


def _wpool(g3, w):
    acc = g3[:, 0, :] * w[:, 0:1]
    for i in range(1, K1):
        acc = jnp.maximum(acc, g3[:, i, :] * w[:, i:i + 1])
    return acc


def _tc_b_body(g3i_ref, g3p_ref, w_ref, ifa_ref, pfa_ref, di_ref, dp_ref):
    w = w_ref[...]
    di_ref[...] = ifa_ref[...] - _wpool(g3i_ref[...], w)
    dp_ref[...] = pfa_ref[...] - _wpool(g3p_ref[...], w)


def _tc_c_body(g3i_ref, g3p_ref, w_ref, di_ref, dp_ref, g_ref,
               tfe_ref, cfe_ref,
               wfc1_ref, bfc1_ref, wfc2_ref, bfc2_ref, wfu_ref, bfu_ref,
               wp1_ref, bp1_ref, wp2_ref, bp2_ref, wp3_ref, bp3_ref,
               out_ref):
    w = w_ref[...]
    di = di_ref[...]
    dp = dp_ref[...]
    pid = _wpool(g3i_ref[...], w)
    ppd = _wpool(g3p_ref[...], w)
    dd = dict(_DOT)
    fuse_i = lax.dot_general(jnp.concatenate([di, ppd], 1), wfc1_ref[...],
                             (((1,), (0,)), ((), ())), **dd) + bfc1_ref[...]
    fuse_p = lax.dot_general(jnp.concatenate([dp, pid], 1), wfc2_ref[...],
                             (((1,), (0,)), ((), ())), **dd) + bfc2_ref[...]
    ft = lax.dot_general(jnp.concatenate([fuse_p, fuse_i], 1), wfu_ref[...],
                         (((1,), (0,)), ((), ())), **dd) + bfu_ref[...]
    x = lax.dot_general(ft, wp1_ref[...], (((1,), (0,)), ((), ())), **dd) + bp1_ref[...]
    h = _lrelu(lax.dot_general(x, wp2_ref[...], (((1,), (0,)), ((), ())), **dd) + bp2_ref[...])
    x = lax.dot_general(h, wp3_ref[...], (((1,), (0,)), ((), ())), **dd) + bp3_ref[...]
    tf = tfe_ref[...] * x
    out_ref[...] = cfe_ref[...] + 0.25 * lax.dot_general(
        g_ref[...], tf, (((1,), (0,)), ((), ())), **dd)


def _pad_pts(p, val):
    p = jnp.pad(p, ((0, NPTS - N), (0, 0)), constant_values=val)
    return jnp.pad(p, ((0, 0), (0, 5)))


def _pad_rows(x):
    return jnp.pad(x, ((0, NPTS - N), (0, 0)))


def kernel(img, cloud, img_tar, cloud_tar, current_feat, target_feat,
           w_conv1, b_conv1, w_conv2, b_conv2, w_pconv1, b_pconv1,
           w_pconv2, b_pconv2, w_fc1, b_fc1, w_fc2, b_fc2, w_fuse2, b_fuse2,
           w_pn1, b_pn1, w_pn2, b_pn2, w_pn3, b_pn3):
    f32 = jnp.float32
    pa = _pad_pts(cloud_tar[0], -1.0e4)
    pb = _pad_pts(cloud[0], 1.0e4)
    xa = _pad_rows(img_tar[0].T)
    xb = _pad_rows(img[0].T)
    tfe = _pad_rows(target_feat[0].T)
    cfe = _pad_rows(current_feat[0].T)

    sds = jax.ShapeDtypeStruct
    tc_a = pl.pallas_call(
        _tc_a_body,
        out_shape=(
            sds((NPTS, 128), f32), sds((NPTS, 128), f32),
            sds((NPTS, 128), f32), sds((NPTS, 128), f32),
            sds((NPTS, K1), jnp.int32), sds((NPTS, K1), f32),
            sds((NPTS, K1), jnp.int32), sds((NPTS, K1), f32),
            sds((NPTS, NPTS), f32),
        ),
    )
    ifa, ifb, pfa, pfb, idx1, w1, idx2, w2, g = tc_a(
        pa, pb, xa, xb,
        w_conv1.T, b_conv1[None], w_conv2.T, b_conv2[None],
        w_pconv1.T, b_pconv1[None], w_pconv2.T, b_pconv2[None])

    g1i, g1p = _sc_gather()(ifb, pfb, idx1.reshape(-1))

    tc_b = pl.pallas_call(
        _tc_b_body,
        out_shape=(sds((NPTS, 128), f32), sds((NPTS, 128), f32)),
    )
    di, dp = tc_b(g1i.reshape(NPTS, K1, 128), g1p.reshape(NPTS, K1, 128),
                  w1, ifa, pfa)

    g2i, g2p = _sc_gather()(di, dp, idx2.reshape(-1))

    tc_c = pl.pallas_call(
        _tc_c_body,
        out_shape=sds((NPTS, 160), f32),
    )
    final = tc_c(g2i.reshape(NPTS, K1, 128), g2p.reshape(NPTS, K1, 128),
                 w2, di, dp, g, tfe, cfe,
                 w_fc1.T, b_fc1[None], w_fc2.T, b_fc2[None],
                 w_fuse2.T, b_fuse2[None],
                 w_pn1.T, b_pn1[None], w_pn2.T, b_pn2[None],
                 w_pn3.T, b_pn3[None])
    return final[:N].T[None]


# speedup vs baseline: 2.7544x; 1.1538x over previous
import functools

import jax
import jax.numpy as jnp
from jax import lax
from jax.experimental import pallas as pl
from jax.experimental.pallas import tpu as pltpu
from jax.experimental.pallas import tpu_sc as plsc

N = 500
NPTS = 512
K1 = 12
KP = 4
NEG = -3.0e38
PAD = NPTS - N

_NC = 2
_NS = 16
_NW = _NC * _NS
_IPW = NPTS * K1 // _NW
_HALF = _IPW // 2

_HI = dict(precision=lax.Precision.HIGHEST, preferred_element_type=jnp.float32)
_MM = dict(precision=lax.Precision.HIGHEST, preferred_element_type=jnp.float32)


def _lrelu(x):
    return jnp.where(x >= 0, x, 0.01 * x)


def _topk(d, k):
    iota = lax.broadcasted_iota(jnp.int32, (NPTS, NPTS), 1)
    idxs, vals = [], []
    work = d
    for _ in range(k):
        m = jnp.min(work, axis=1, keepdims=True)
        a = jnp.min(jnp.where(work == m, iota, NPTS), axis=1, keepdims=True)
        idxs.append(a)
        vals.append(m)
        work = jnp.where(iota == a, 3.0e38, work)
    return jnp.concatenate(idxs, 1), jnp.concatenate(vals, 1)


def _softw(vals):
    nrm = jnp.sqrt(jnp.maximum(vals, 1e-12))
    rmask = lax.broadcasted_iota(jnp.int32, (NPTS, K1), 0) < N
    logit = jnp.where(rmask, -nrm, NEG)
    e = jnp.exp(logit - jnp.max(logit))
    return e / jnp.sum(e)


def _tc_a_body(pa_ref, pb_ref, xa_ref, xb_ref,
               w1c_ref, b1c_ref, w2c_ref, b2c_ref,
               w1p_ref, b1p_ref, w2p_ref, b2p_ref,
               ifa_ref, ifb_ref, pfa_ref, pfb_ref,
               idx1_ref, w1_ref, idx2_ref, w2_ref, g_ref):
    pa = jnp.concatenate(
        [pa_ref[...], jnp.full((PAD, 3), -1.0e4, jnp.float32)], 0)
    pb = jnp.concatenate(
        [pb_ref[...], jnp.full((PAD, 3), 1.0e4, jnp.float32)], 0)
    sa = jnp.sum(pa * pa, axis=1, keepdims=True)
    sb = jnp.sum(pb * pb, axis=1, keepdims=True)
    one = jnp.ones((NPTS, 1), jnp.float32)
    ua = jnp.concatenate([-2.0 * pa, sa, one], 1)
    ub = jnp.concatenate([-2.0 * pb, sb, one], 1)
    va = jnp.concatenate([pa, one, sa], 1)
    vb = jnp.concatenate([pb, one, sb], 1)
    dn = (((1,), (1,)), ((), ()))
    d1 = lax.dot_general(ua, vb, dn, **_HI)
    d2 = lax.dot_general(ua, va, dn, **_HI)
    d1t = lax.dot_general(ub, va, dn, **_HI)

    idx1, v1 = _topk(d1, K1)
    idx2, v2 = _topk(d2, K1)
    idxp, _ = _topk(d1t, KP)

    idx1_ref[...] = idx1
    idx2_ref[...] = idx2
    w1_ref[...] = _softw(v1)
    w2_ref[...] = _softw(v2)

    giota = lax.broadcasted_iota(jnp.int32, (NPTS, NPTS), 1)
    g = jnp.zeros((NPTS, NPTS), jnp.float32)
    for i in range(KP):
        g = g + (idxp[:, i:i + 1] == giota).astype(jnp.float32)
    g_ref[...] = g

    def mlp_img(x):
        x = jnp.concatenate([x, jnp.zeros((32, PAD), jnp.float32)], 1)
        h = lax.dot_general(x, w1c_ref[...], (((0,), (1,)), ((), ())), **_MM)
        h = _lrelu(h + b1c_ref[...])
        return lax.dot_general(h, w2c_ref[...], (((1,), (1,)), ((), ())),
                               **_MM) + b2c_ref[...]

    def mlp_pts(p):
        h = lax.dot_general(p, w1p_ref[...], (((1,), (1,)), ((), ())), **_MM)
        h = _lrelu(h + b1p_ref[...])
        return lax.dot_general(h, w2p_ref[...], (((1,), (1,)), ((), ())),
                               **_MM) + b2p_ref[...]

    ifa_ref[...] = mlp_img(xa_ref[...])
    ifb_ref[...] = mlp_img(xb_ref[...])
    pfa_ref[...] = mlp_pts(pa)
    pfb_ref[...] = mlp_pts(pb)


def _sc_gather_body(taba_ref, tabb_ref, idx_ref,
                    outa_ref, outb_ref,
                    idxv0_ref, idxv1_ref,
                    bufa0_ref, bufa1_ref, bufb0_ref, bufb1_ref,
                    gsem, wsem):
    wid = lax.axis_index("s") * _NC + lax.axis_index("c")
    base = wid * _IPW
    pltpu.sync_copy(idx_ref.at[pl.ds(base, _HALF)], idxv0_ref)
    pltpu.sync_copy(idx_ref.at[pl.ds(base + _HALF, _HALF)], idxv1_ref)
    ga0 = pltpu.async_copy(taba_ref.at[idxv0_ref], bufa0_ref, gsem)
    ga1 = pltpu.async_copy(taba_ref.at[idxv1_ref], bufa1_ref, gsem)
    gb0 = pltpu.async_copy(tabb_ref.at[idxv0_ref], bufb0_ref, gsem)
    gb1 = pltpu.async_copy(tabb_ref.at[idxv1_ref], bufb1_ref, gsem)
    ga0.wait()
    wa0 = pltpu.async_copy(bufa0_ref, outa_ref.at[pl.ds(base, _HALF)], wsem)
    ga1.wait()
    wa1 = pltpu.async_copy(bufa1_ref, outa_ref.at[pl.ds(base + _HALF, _HALF)],
                           wsem)
    gb0.wait()
    wb0 = pltpu.async_copy(bufb0_ref, outb_ref.at[pl.ds(base, _HALF)], wsem)
    gb1.wait()
    wb1 = pltpu.async_copy(bufb1_ref, outb_ref.at[pl.ds(base + _HALF, _HALF)],
                           wsem)
    wa0.wait()
    wa1.wait()
    wb0.wait()
    wb1.wait()


@functools.cache
def _sc_gather():
    return functools.partial(
        pl.kernel,
        out_type=[jax.ShapeDtypeStruct((NPTS * K1, 128), jnp.float32)] * 2,
        mesh=plsc.VectorSubcoreMesh(core_axis_name="c", subcore_axis_name="s",
                                    num_cores=_NC, num_subcores=_NS),
        scratch_types=[
            pltpu.VMEM((_HALF,), jnp.int32),
            pltpu.VMEM((_HALF,), jnp.int32),
            pltpu.VMEM((_HALF, 128), jnp.float32),
            pltpu.VMEM((_HALF, 128), jnp.float32),
            pltpu.VMEM((_HALF, 128), jnp.float32),
            pltpu.VMEM((_HALF, 128), jnp.float32),
            pltpu.SemaphoreType.DMA,
            pltpu.SemaphoreType.DMA,
        ],
    )(_sc_gather_body)


def _wpool(g2, w):
    acc = g2[:, 0:128] * w[:, 0:1]
    for i in range(1, K1):
        acc = jnp.maximum(acc, g2[:, i * 128:(i + 1) * 128] * w[:, i:i + 1])
    return acc


def _tc_b_body(g2i_ref, g2p_ref, w_ref, ifa_ref, pfa_ref, di_ref, dp_ref):
    w = w_ref[...]
    di_ref[...] = ifa_ref[...] - _wpool(g2i_ref[...], w)
    dp_ref[...] = pfa_ref[...] - _wpool(g2p_ref[...], w)


def _tc_c_body(g2i_ref, g2p_ref, w_ref, di_ref, dp_ref, g_ref,
               tfe_ref, cfe_ref,
               wfc1_ref, bfc1_ref, wfc2_ref, bfc2_ref, wfu_ref, bfu_ref,
               wp1_ref, bp1_ref, wp2_ref, bp2_ref, wp3_ref, bp3_ref,
               out_ref):
    w = w_ref[...]
    di = di_ref[...]
    dp = dp_ref[...]
    pid = _wpool(g2i_ref[...], w)
    ppd = _wpool(g2p_ref[...], w)
    ct = (((1,), (1,)), ((), ()))
    cm = (((1,), (0,)), ((), ()))
    fuse_i = lax.dot_general(wfc1_ref[...], jnp.concatenate([di, ppd], 1),
                             ct, **_MM) + bfc1_ref[...]
    fuse_p = lax.dot_general(wfc2_ref[...], jnp.concatenate([dp, pid], 1),
                             ct, **_MM) + bfc2_ref[...]
    ft = lax.dot_general(wfu_ref[...], jnp.concatenate([fuse_p, fuse_i], 0),
                         cm, **_MM) + bfu_ref[...]
    x = lax.dot_general(wp1_ref[...], ft, cm, **_MM) + bp1_ref[...]
    h = _lrelu(lax.dot_general(wp2_ref[...], x, cm, **_MM) + bp2_ref[...])
    x = lax.dot_general(wp3_ref[...], h, cm, **_MM) + bp3_ref[...]
    zpad = jnp.zeros((160, PAD), jnp.float32)
    tfx = jnp.concatenate([tfe_ref[...], zpad], 1) * x
    out_ref[...] = jnp.concatenate([cfe_ref[...], zpad], 1) + 0.25 * \
        lax.dot_general(tfx, g_ref[...], (((1,), (1,)), ((), ())), **_MM)


def kernel(img, cloud, img_tar, cloud_tar, current_feat, target_feat,
           w_conv1, b_conv1, w_conv2, b_conv2, w_pconv1, b_pconv1,
           w_pconv2, b_pconv2, w_fc1, b_fc1, w_fc2, b_fc2, w_fuse2, b_fuse2,
           w_pn1, b_pn1, w_pn2, b_pn2, w_pn3, b_pn3):
    f32 = jnp.float32
    sds = jax.ShapeDtypeStruct
    tc_a = pl.pallas_call(
        _tc_a_body,
        out_shape=(
            sds((NPTS, 128), f32), sds((NPTS, 128), f32),
            sds((NPTS, 128), f32), sds((NPTS, 128), f32),
            sds((NPTS, K1), jnp.int32), sds((NPTS, K1), f32),
            sds((NPTS, K1), jnp.int32), sds((NPTS, K1), f32),
            sds((NPTS, NPTS), f32),
        ),
    )
    ifa, ifb, pfa, pfb, idx1, w1, idx2, w2, g = tc_a(
        cloud_tar[0], cloud[0], img_tar[0], img[0],
        w_conv1, b_conv1[None], w_conv2, b_conv2[None],
        w_pconv1, b_pconv1[None], w_pconv2, b_pconv2[None])

    g1i, g1p = _sc_gather()(ifb, pfb, idx1.reshape(-1))

    tc_b = pl.pallas_call(
        _tc_b_body,
        out_shape=(sds((NPTS, 128), f32), sds((NPTS, 128), f32)),
    )
    di, dp = tc_b(g1i.reshape(NPTS, K1 * 128), g1p.reshape(NPTS, K1 * 128),
                  w1, ifa, pfa)

    g2i, g2p = _sc_gather()(di, dp, idx2.reshape(-1))

    tc_c = pl.pallas_call(
        _tc_c_body,
        out_shape=sds((160, NPTS), f32),
    )
    final = tc_c(g2i.reshape(NPTS, K1 * 128), g2p.reshape(NPTS, K1 * 128),
                 w2, di, dp, g, target_feat[0], current_feat[0],
                 w_fc1, b_fc1[:, None], w_fc2, b_fc2[:, None],
                 w_fuse2, b_fuse2[:, None],
                 w_pn1, b_pn1[:, None], w_pn2, b_pn2[:, None],
                 w_pn3, b_pn3[:, None])
    return final[None, :, :N]


# speedup vs baseline: 3.5306x; 1.2818x over previous
import functools

import jax
import jax.numpy as jnp
from jax import lax
from jax.experimental import pallas as pl
from jax.experimental.pallas import tpu as pltpu
from jax.experimental.pallas import tpu_sc as plsc

N = 500
NPTS = 512
K1 = 12
KP = 4
NEG = -3.0e38
PAD = NPTS - N

_NC = 2
_NS = 16
_NW = _NC * _NS
_QPW = NPTS // _NW
_IPW = _QPW * K1
_HALF = _IPW // 2
_QH = _HALF // K1

_HI = dict(precision=lax.Precision.HIGHEST, preferred_element_type=jnp.float32)


def _lrelu(x):
    return jnp.where(x >= 0, x, 0.01 * x)


def _topk(d, k):
    iota = lax.broadcasted_iota(jnp.int32, (NPTS, NPTS), 1)
    idxs, vals = [], []
    work = d
    for _ in range(k):
        m = jnp.min(work, axis=1, keepdims=True)
        a = jnp.min(jnp.where(work == m, iota, NPTS), axis=1, keepdims=True)
        idxs.append(a)
        vals.append(m)
        work = jnp.where(iota == a, 3.0e38, work)
    return jnp.concatenate(idxs, 1), jnp.concatenate(vals, 1)


def _softw(vals):
    nrm = jnp.sqrt(jnp.maximum(vals, 1e-12))
    rmask = lax.broadcasted_iota(jnp.int32, (NPTS, K1), 0) < N
    logit = jnp.where(rmask, -nrm, NEG)
    e = jnp.exp(logit - jnp.max(logit))
    return e / jnp.sum(e)


def _tc_a_body(pa_ref, pb_ref, xa_ref, xb_ref,
               w1c_ref, b1c_ref, w2c_ref, b2c_ref,
               w1p_ref, b1p_ref, w2p_ref, b2p_ref,
               ifa_ref, ifb_ref, pfa_ref, pfb_ref,
               idx1_ref, w1_ref, idx2_ref, w2_ref, g_ref):
    pa = jnp.concatenate(
        [pa_ref[...], jnp.full((PAD, 3), -1.0e4, jnp.float32)], 0)
    pb = jnp.concatenate(
        [pb_ref[...], jnp.full((PAD, 3), 1.0e4, jnp.float32)], 0)
    sa = jnp.sum(pa * pa, axis=1, keepdims=True)
    sb = jnp.sum(pb * pb, axis=1, keepdims=True)
    one = jnp.ones((NPTS, 1), jnp.float32)
    ua = jnp.concatenate([-2.0 * pa, sa, one], 1)
    ub = jnp.concatenate([-2.0 * pb, sb, one], 1)
    va = jnp.concatenate([pa, one, sa], 1)
    vb = jnp.concatenate([pb, one, sb], 1)
    dn = (((1,), (1,)), ((), ()))
    d1 = lax.dot_general(ua, vb, dn, **_HI)
    d2 = lax.dot_general(ua, va, dn, **_HI)
    d1t = lax.dot_general(ub, va, dn, **_HI)

    idx1, v1 = _topk(d1, K1)
    idx2, v2 = _topk(d2, K1)
    idxp, _ = _topk(d1t, KP)

    idx1_ref[...] = idx1
    idx2_ref[...] = idx2
    w1_ref[...] = _softw(v1)
    w2_ref[...] = _softw(v2)

    giota = lax.broadcasted_iota(jnp.int32, (NPTS, NPTS), 1)
    g = jnp.zeros((NPTS, NPTS), jnp.float32)
    for i in range(KP):
        g = g + (idxp[:, i:i + 1] == giota).astype(jnp.float32)
    g_ref[...] = g

    def mlp_img(x):
        x = jnp.concatenate([x, jnp.zeros((32, PAD), jnp.float32)], 1)
        h = lax.dot_general(x, w1c_ref[...], (((0,), (1,)), ((), ())), **_HI)
        h = _lrelu(h + b1c_ref[...])
        return lax.dot_general(h, w2c_ref[...], (((1,), (1,)), ((), ())),
                               **_HI) + b2c_ref[...]

    def mlp_pts(p):
        h = lax.dot_general(p, w1p_ref[...], (((1,), (1,)), ((), ())), **_HI)
        h = _lrelu(h + b1p_ref[...])
        return lax.dot_general(h, w2p_ref[...], (((1,), (1,)), ((), ())),
                               **_HI) + b2p_ref[...]

    ifa_ref[...] = mlp_img(xa_ref[...])
    ifb_ref[...] = mlp_img(xb_ref[...])
    pfa_ref[...] = mlp_pts(pa)
    pfb_ref[...] = mlp_pts(pb)


def _sc_pool_halves(wv_ref, buf0, buf1, refv, outv, q0, diff):
    def make_body(buf, qoff):
        def body(q, c):
            r0 = (q - qoff) * K1
            wq = wv_ref[pl.ds(q * K1, 16)]
            ws = [wq[i] for i in range(K1)]
            for j in range(8):
                sl = pl.ds(j * 16, 16)
                acc = buf[r0, sl] * ws[0]
                for i in range(1, K1):
                    acc = jnp.maximum(acc, buf[r0 + i, sl] * ws[i])
                if diff:
                    outv[q, sl] = refv[q, sl] - acc
                else:
                    outv[q, sl] = acc
            return c
        return body
    lax.fori_loop(0, _QH, make_body(buf0, 0), 0)
    lax.fori_loop(_QH, 2 * _QH, make_body(buf1, _QH), 0)
    del q0


def _sc_stage_body(diff, taba_ref, tabb_ref, idx_ref, w_ref, refa_ref, refb_ref,
                   outa_ref, outb_ref,
                   idxv0, idxv1, wv,
                   bufa0, bufa1, bufb0, bufb1,
                   refav, refbv, outav, outbv, gsem):
    wid = lax.axis_index("s") * _NC + lax.axis_index("c")
    base = wid * _IPW
    qb = wid * _QPW
    pltpu.sync_copy(idx_ref.at[pl.ds(base, _HALF)], idxv0)
    pltpu.sync_copy(idx_ref.at[pl.ds(base + _HALF, _HALF)], idxv1)
    ga0 = pltpu.async_copy(taba_ref.at[idxv0], bufa0, gsem)
    ga1 = pltpu.async_copy(taba_ref.at[idxv1], bufa1, gsem)
    gb0 = pltpu.async_copy(tabb_ref.at[idxv0], bufb0, gsem)
    gb1 = pltpu.async_copy(tabb_ref.at[idxv1], bufb1, gsem)
    pltpu.sync_copy(w_ref.at[pl.ds(base, _IPW)], wv.at[pl.ds(0, _IPW)])
    if diff:
        pltpu.sync_copy(refa_ref.at[pl.ds(qb, _QPW)], refav)
        pltpu.sync_copy(refb_ref.at[pl.ds(qb, _QPW)], refbv)
    ga0.wait()
    ga1.wait()
    gb0.wait()
    gb1.wait()
    _sc_pool_halves(wv, bufa0, bufa1, refav, outav, qb, diff)
    _sc_pool_halves(wv, bufb0, bufb1, refbv, outbv, qb, diff)
    pltpu.sync_copy(outav, outa_ref.at[pl.ds(qb, _QPW)])
    pltpu.sync_copy(outbv, outb_ref.at[pl.ds(qb, _QPW)])


@functools.cache
def _sc_stage(diff):
    return functools.partial(
        pl.kernel,
        out_type=[jax.ShapeDtypeStruct((NPTS, 128), jnp.float32)] * 2,
        mesh=plsc.VectorSubcoreMesh(core_axis_name="c", subcore_axis_name="s",
                                    num_cores=_NC, num_subcores=_NS),
        scratch_types=[
            pltpu.VMEM((_HALF,), jnp.int32),
            pltpu.VMEM((_HALF,), jnp.int32),
            pltpu.VMEM((_IPW + 16,), jnp.float32),
            pltpu.VMEM((_HALF, 128), jnp.float32),
            pltpu.VMEM((_HALF, 128), jnp.float32),
            pltpu.VMEM((_HALF, 128), jnp.float32),
            pltpu.VMEM((_HALF, 128), jnp.float32),
            pltpu.VMEM((_QPW, 128), jnp.float32),
            pltpu.VMEM((_QPW, 128), jnp.float32),
            pltpu.VMEM((_QPW, 128), jnp.float32),
            pltpu.VMEM((_QPW, 128), jnp.float32),
            pltpu.SemaphoreType.DMA,
        ],
    )(functools.partial(_sc_stage_body, diff))


def _tc_c_body(pid_ref, ppd_ref, di_ref, dp_ref, g_ref,
               tfe_ref, cfe_ref,
               wfc1_ref, bfc1_ref, wfc2_ref, bfc2_ref, wfu_ref, bfu_ref,
               wp1_ref, bp1_ref, wp2_ref, bp2_ref, wp3_ref, bp3_ref,
               out_ref):
    di = di_ref[...]
    dp = dp_ref[...]
    pid = pid_ref[...]
    ppd = ppd_ref[...]
    ct = (((1,), (1,)), ((), ()))
    cm = (((1,), (0,)), ((), ()))
    fuse_i = lax.dot_general(wfc1_ref[...], jnp.concatenate([di, ppd], 1),
                             ct, **_HI) + bfc1_ref[...]
    fuse_p = lax.dot_general(wfc2_ref[...], jnp.concatenate([dp, pid], 1),
                             ct, **_HI) + bfc2_ref[...]
    ft = lax.dot_general(wfu_ref[...], jnp.concatenate([fuse_p, fuse_i], 0),
                         cm, **_HI) + bfu_ref[...]
    x = lax.dot_general(wp1_ref[...], ft, cm, **_HI) + bp1_ref[...]
    h = _lrelu(lax.dot_general(wp2_ref[...], x, cm, **_HI) + bp2_ref[...])
    x = lax.dot_general(wp3_ref[...], h, cm, **_HI) + bp3_ref[...]
    zpad = jnp.zeros((160, PAD), jnp.float32)
    tfx = jnp.concatenate([tfe_ref[...], zpad], 1) * x
    out_ref[...] = jnp.concatenate([cfe_ref[...], zpad], 1) + 0.25 * \
        lax.dot_general(tfx, g_ref[...], (((1,), (1,)), ((), ())), **_HI)


def kernel(img, cloud, img_tar, cloud_tar, current_feat, target_feat,
           w_conv1, b_conv1, w_conv2, b_conv2, w_pconv1, b_pconv1,
           w_pconv2, b_pconv2, w_fc1, b_fc1, w_fc2, b_fc2, w_fuse2, b_fuse2,
           w_pn1, b_pn1, w_pn2, b_pn2, w_pn3, b_pn3):
    f32 = jnp.float32
    sds = jax.ShapeDtypeStruct
    tc_a = pl.pallas_call(
        _tc_a_body,
        out_shape=(
            sds((NPTS, 128), f32), sds((NPTS, 128), f32),
            sds((NPTS, 128), f32), sds((NPTS, 128), f32),
            sds((NPTS, K1), jnp.int32), sds((NPTS, K1), f32),
            sds((NPTS, K1), jnp.int32), sds((NPTS, K1), f32),
            sds((NPTS, NPTS), f32),
        ),
    )
    ifa, ifb, pfa, pfb, idx1, w1, idx2, w2, g = tc_a(
        cloud_tar[0], cloud[0], img_tar[0], img[0],
        w_conv1, b_conv1[None], w_conv2, b_conv2[None],
        w_pconv1, b_pconv1[None], w_pconv2, b_pconv2[None])

    di, dp = _sc_stage(True)(ifb, pfb, idx1.reshape(-1), w1.reshape(-1),
                             ifa, pfa)
    pid, ppd = _sc_stage(False)(di, dp, idx2.reshape(-1), w2.reshape(-1),
                                di, dp)

    tc_c = pl.pallas_call(
        _tc_c_body,
        out_shape=sds((160, NPTS), f32),
    )
    final = tc_c(pid, ppd, di, dp, g, target_feat[0], current_feat[0],
                 w_fc1, b_fc1[:, None], w_fc2, b_fc2[:, None],
                 w_fuse2, b_fuse2[:, None],
                 w_pn1, b_pn1[:, None], w_pn2, b_pn2[:, None],
                 w_pn3, b_pn3[:, None])
    return final[None, :, :N]


# speedup vs baseline: 3.6029x; 1.0205x over previous
import functools

import jax
import jax.numpy as jnp
from jax import lax
from jax.experimental import pallas as pl
from jax.experimental.pallas import tpu as pltpu
from jax.experimental.pallas import tpu_sc as plsc

N = 500
NPTS = 512
K1 = 12
KP = 4
NEG = -3.0e38
PAD = NPTS - N

_NC = 2
_NS = 16
_NW = _NC * _NS
_QPW = NPTS // _NW
_IPW = _QPW * K1
_HALF = _IPW // 2
_QH = _HALF // K1

_HI = dict(precision=lax.Precision.HIGHEST, preferred_element_type=jnp.float32)


def _lrelu(x):
    return jnp.where(x >= 0, x, 0.01 * x)


def _topk(d, k):
    iota = lax.broadcasted_iota(jnp.int32, (NPTS, NPTS), 1)
    idxs, vals = [], []
    work = d
    for _ in range(k):
        m = jnp.min(work, axis=1, keepdims=True)
        a = jnp.min(jnp.where(work == m, iota, NPTS), axis=1, keepdims=True)
        idxs.append(a)
        vals.append(m)
        work = jnp.where(iota == a, 3.0e38, work)
    return jnp.concatenate(idxs, 1), jnp.concatenate(vals, 1)


def _softw(vals):
    nrm = jnp.sqrt(jnp.maximum(vals, 1e-12))
    rmask = lax.broadcasted_iota(jnp.int32, (NPTS, K1), 0) < N
    logit = jnp.where(rmask, -nrm, NEG)
    e = jnp.exp(logit - jnp.max(logit))
    return e / jnp.sum(e)


def _tc_a_body(pa_ref, pb_ref, xa_ref, xb_ref,
               w1c_ref, b1c_ref, w2c_ref, b2c_ref,
               w1p_ref, b1p_ref, w2p_ref, b2p_ref,
               ifa_ref, ifb_ref, pfa_ref, pfb_ref,
               idx1_ref, w1_ref, idx2_ref, w2_ref, g_ref):
    pa = jnp.concatenate(
        [pa_ref[...], jnp.full((PAD, 3), -1.0e4, jnp.float32)], 0)
    pb = jnp.concatenate(
        [pb_ref[...], jnp.full((PAD, 3), 1.0e4, jnp.float32)], 0)
    sa = jnp.sum(pa * pa, axis=1, keepdims=True)
    sb = jnp.sum(pb * pb, axis=1, keepdims=True)
    one = jnp.ones((NPTS, 1), jnp.float32)
    ua = jnp.concatenate([-2.0 * pa, sa, one], 1)
    ub = jnp.concatenate([-2.0 * pb, sb, one], 1)
    va = jnp.concatenate([pa, one, sa], 1)
    vb = jnp.concatenate([pb, one, sb], 1)
    dn = (((1,), (1,)), ((), ()))
    d1 = lax.dot_general(ua, vb, dn, **_HI)
    d2 = lax.dot_general(ua, va, dn, **_HI)
    d1t = lax.dot_general(ub, va, dn, **_HI)

    idx1, v1 = _topk(d1, K1)
    idx2, v2 = _topk(d2, K1)
    idxp, _ = _topk(d1t, KP)

    idx1_ref[...] = idx1
    idx2_ref[...] = idx2
    w1_ref[...] = _softw(v1)
    w2_ref[...] = _softw(v2)

    giota = lax.broadcasted_iota(jnp.int32, (NPTS, NPTS), 1)
    g = jnp.zeros((NPTS, NPTS), jnp.float32)
    for i in range(KP):
        g = g + (idxp[:, i:i + 1] == giota).astype(jnp.float32)
    g_ref[...] = g

    def mlp_img(x):
        x = jnp.concatenate([x, jnp.zeros((32, PAD), jnp.float32)], 1)
        h = lax.dot_general(x, w1c_ref[...], (((0,), (1,)), ((), ())), **_HI)
        h = _lrelu(h + b1c_ref[...])
        return lax.dot_general(h, w2c_ref[...], (((1,), (1,)), ((), ())),
                               **_HI) + b2c_ref[...]

    def mlp_pts(p):
        h = lax.dot_general(p, w1p_ref[...], (((1,), (1,)), ((), ())), **_HI)
        h = _lrelu(h + b1p_ref[...])
        return lax.dot_general(h, w2p_ref[...], (((1,), (1,)), ((), ())),
                               **_HI) + b2p_ref[...]

    ifa_ref[...] = mlp_img(xa_ref[...])
    ifb_ref[...] = mlp_img(xb_ref[...])
    pfa_ref[...] = mlp_pts(pa)
    pfb_ref[...] = mlp_pts(pb)


def _sc_pool_half(wv_ref, buf, refv, outv, qoff, diff):
    def body(q, c):
        r0 = (q - qoff) * K1
        wq = wv_ref[pl.ds(q * K1, 16)]
        ws = [wq[i] for i in range(K1)]
        for j in range(8):
            sl = pl.ds(j * 16, 16)
            acc = buf[r0, sl] * ws[0]
            for i in range(1, K1):
                acc = jnp.maximum(acc, buf[r0 + i, sl] * ws[i])
            if diff:
                outv[q, sl] = refv[q, sl] - acc
            else:
                outv[q, sl] = acc
        return c
    lax.fori_loop(qoff, qoff + _QH, body, 0)


def _sc_stage_body(diff, taba_ref, tabb_ref, idx_ref, w_ref, refa_ref, refb_ref,
                   outa_ref, outb_ref,
                   idxv, wv,
                   bufa0, bufa1, bufb0, bufb1,
                   refav, refbv, outav, outbv, gsem, wsem):
    wid = lax.axis_index("s") * _NC + lax.axis_index("c")
    base = wid * _IPW
    qb = wid * _QPW
    pltpu.sync_copy(idx_ref.at[pl.ds(base, _IPW)], idxv)
    i0 = idxv.at[pl.ds(0, _HALF)]
    i1 = idxv.at[pl.ds(_HALF, _HALF)]
    ga0 = pltpu.async_copy(taba_ref.at[i0], bufa0, gsem)
    ga1 = pltpu.async_copy(taba_ref.at[i1], bufa1, gsem)
    gb0 = pltpu.async_copy(tabb_ref.at[i0], bufb0, gsem)
    gb1 = pltpu.async_copy(tabb_ref.at[i1], bufb1, gsem)
    pltpu.sync_copy(w_ref.at[pl.ds(base, _IPW)], wv.at[pl.ds(0, _IPW)])
    if diff:
        pltpu.sync_copy(refa_ref.at[pl.ds(qb, _QPW)], refav)
        pltpu.sync_copy(refb_ref.at[pl.ds(qb, _QPW)], refbv)
    ga0.wait()
    _sc_pool_half(wv, bufa0, refav, outav, 0, diff)
    ga1.wait()
    _sc_pool_half(wv, bufa1, refav, outav, _QH, diff)
    wa = pltpu.async_copy(outav, outa_ref.at[pl.ds(qb, _QPW)], wsem)
    gb0.wait()
    _sc_pool_half(wv, bufb0, refbv, outbv, 0, diff)
    gb1.wait()
    _sc_pool_half(wv, bufb1, refbv, outbv, _QH, diff)
    wb = pltpu.async_copy(outbv, outb_ref.at[pl.ds(qb, _QPW)], wsem)
    wa.wait()
    wb.wait()


@functools.cache
def _sc_stage(diff):
    return functools.partial(
        pl.kernel,
        out_type=[jax.ShapeDtypeStruct((NPTS, 128), jnp.float32)] * 2,
        mesh=plsc.VectorSubcoreMesh(core_axis_name="c", subcore_axis_name="s",
                                    num_cores=_NC, num_subcores=_NS),
        scratch_types=[
            pltpu.VMEM((_IPW,), jnp.int32),
            pltpu.VMEM((_IPW + 16,), jnp.float32),
            pltpu.VMEM((_HALF, 128), jnp.float32),
            pltpu.VMEM((_HALF, 128), jnp.float32),
            pltpu.VMEM((_HALF, 128), jnp.float32),
            pltpu.VMEM((_HALF, 128), jnp.float32),
            pltpu.VMEM((_QPW, 128), jnp.float32),
            pltpu.VMEM((_QPW, 128), jnp.float32),
            pltpu.VMEM((_QPW, 128), jnp.float32),
            pltpu.VMEM((_QPW, 128), jnp.float32),
            pltpu.SemaphoreType.DMA,
            pltpu.SemaphoreType.DMA,
        ],
    )(functools.partial(_sc_stage_body, diff))


def _tc_c_body(pid_ref, ppd_ref, di_ref, dp_ref, g_ref,
               tfe_ref, cfe_ref,
               wfc1_ref, bfc1_ref, wfc2_ref, bfc2_ref, wfu_ref, bfu_ref,
               wp1_ref, bp1_ref, wp2_ref, bp2_ref, wp3_ref, bp3_ref,
               out_ref):
    di = di_ref[...]
    dp = dp_ref[...]
    pid = pid_ref[...]
    ppd = ppd_ref[...]
    ct = (((1,), (1,)), ((), ()))
    cm = (((1,), (0,)), ((), ()))
    fuse_i = lax.dot_general(wfc1_ref[...], jnp.concatenate([di, ppd], 1),
                             ct, **_HI) + bfc1_ref[...]
    fuse_p = lax.dot_general(wfc2_ref[...], jnp.concatenate([dp, pid], 1),
                             ct, **_HI) + bfc2_ref[...]
    ft = lax.dot_general(wfu_ref[...], jnp.concatenate([fuse_p, fuse_i], 0),
                         cm, **_HI) + bfu_ref[...]
    x = lax.dot_general(wp1_ref[...], ft, cm, **_HI) + bp1_ref[...]
    h = _lrelu(lax.dot_general(wp2_ref[...], x, cm, **_HI) + bp2_ref[...])
    x = lax.dot_general(wp3_ref[...], h, cm, **_HI) + bp3_ref[...]
    zpad = jnp.zeros((160, PAD), jnp.float32)
    tfx = jnp.concatenate([tfe_ref[...], zpad], 1) * x
    out_ref[...] = jnp.concatenate([cfe_ref[...], zpad], 1) + 0.25 * \
        lax.dot_general(tfx, g_ref[...], (((1,), (1,)), ((), ())), **_HI)


def kernel(img, cloud, img_tar, cloud_tar, current_feat, target_feat,
           w_conv1, b_conv1, w_conv2, b_conv2, w_pconv1, b_pconv1,
           w_pconv2, b_pconv2, w_fc1, b_fc1, w_fc2, b_fc2, w_fuse2, b_fuse2,
           w_pn1, b_pn1, w_pn2, b_pn2, w_pn3, b_pn3):
    f32 = jnp.float32
    sds = jax.ShapeDtypeStruct
    tc_a = pl.pallas_call(
        _tc_a_body,
        out_shape=(
            sds((NPTS, 128), f32), sds((NPTS, 128), f32),
            sds((NPTS, 128), f32), sds((NPTS, 128), f32),
            sds((NPTS, K1), jnp.int32), sds((NPTS, K1), f32),
            sds((NPTS, K1), jnp.int32), sds((NPTS, K1), f32),
            sds((NPTS, NPTS), f32),
        ),
    )
    ifa, ifb, pfa, pfb, idx1, w1, idx2, w2, g = tc_a(
        cloud_tar[0], cloud[0], img_tar[0], img[0],
        w_conv1, b_conv1[None], w_conv2, b_conv2[None],
        w_pconv1, b_pconv1[None], w_pconv2, b_pconv2[None])

    di, dp = _sc_stage(True)(ifb, pfb, idx1.reshape(-1), w1.reshape(-1),
                             ifa, pfa)
    pid, ppd = _sc_stage(False)(di, dp, idx2.reshape(-1), w2.reshape(-1),
                                di, dp)

    tc_c = pl.pallas_call(
        _tc_c_body,
        out_shape=sds((160, NPTS), f32),
    )
    final = tc_c(pid, ppd, di, dp, g, target_feat[0], current_feat[0],
                 w_fc1, b_fc1[:, None], w_fc2, b_fc2[:, None],
                 w_fuse2, b_fuse2[:, None],
                 w_pn1, b_pn1[:, None], w_pn2, b_pn2[:, None],
                 w_pn3, b_pn3[:, None])
    return final[None, :, :N]
